# baseline (speedup 1.0000x reference)
import jax
import jax.numpy as jnp
from jax import lax
from jax.experimental import pallas as pl
from jax.experimental.pallas import tpu as pltpu
from jax.experimental.pallas import tpu_sc as plsc

N = 10000
E = 320000
D_FEAT = 128
D_TOPO = 64
D_LAT = 128
D_OUT = 128

NC = 2
NS = 16
NW = NC * NS

N_PAD = 10240
STRIPE = N_PAD // NS
WB1 = 128
WINS1 = 80
WB2 = 128
WINS2 = 160
EPT = WB1 * WINS1
E_PAD = NW * EPT
NB1, K1 = 8, 4
NB2, K2 = 5, 3
R_BLK = 1024


def _ring(table, sidx, didx, acc, rbufs, gsems, ssems, wins, nb, k,
          deg_issue=None, deg_drain=None):
    p = nb - k

    def gath(w, b):
        pltpu.async_copy(table.at[sidx.at[w]], rbufs[b], gsems[b])

    def gwait(b):
        pltpu.make_async_copy(table.at[sidx.at[0]], rbufs[b], gsems[b]).wait()

    def scat(w, b):
        pltpu.async_copy(rbufs[b], acc.at[didx.at[w]], ssems[b], add=True)

    def swait(b):
        pltpu.make_async_copy(rbufs[b], acc.at[didx.at[0]], ssems[b]).wait()

    for w in range(p):
        gath(w, w)
    for w in range(k):
        b = w % nb
        gwait(b)
        scat(w, b)
        if deg_issue is not None:
            deg_issue(w)
        gath(w + p, (w + p) % nb)

    def group(g, carry):
        w0 = k + g * nb
        for j in range(nb):
            w = w0 + j
            b = (k + j) % nb
            gwait(b)
            scat(w, b)
            if deg_issue is not None:
                deg_issue(w)
                deg_drain()
            swait(j % nb)
            gath(w + p, j % nb)
        return carry

    lax.fori_loop(0, (wins - nb) // nb, group, 0)
    for t in range(p):
        w = wins - p + t
        b = (k + t) % nb
        gwait(b)
        scat(w, b)
        if deg_issue is not None:
            deg_issue(w)
            deg_drain()
    for b in range(nb):
        swait(b)
    if deg_drain is not None:
        for _ in range(k):
            deg_drain()


def _seg_body_p1(srcg, dstg, topo, z64, zd, outp, degp,
                 sidx, didx, rbufs, obuf, acc, dacc, gsems, ssems, dsem):
    c = lax.axis_index("c")
    s = lax.axis_index("s")
    wid = c * NS + s
    r0 = s * STRIPE
    pltpu.sync_copy(srcg.at[wid], sidx)
    pltpu.sync_copy(dstg.at[wid], didx)
    for i in range(8):
        obuf[pl.ds(i * 16, 16)] = jnp.ones((16,), jnp.float32)
    pltpu.sync_copy(z64.at[pl.ds(r0, STRIPE), :], acc.at[pl.ds(r0, STRIPE), :])
    pltpu.sync_copy(zd.at[pl.ds(r0, STRIPE)], dacc.at[pl.ds(r0, STRIPE)])
    plsc.subcore_barrier()

    def deg_issue(w):
        pltpu.async_copy(obuf, dacc.at[didx.at[w]], dsem, add=True)

    def deg_drain():
        pltpu.make_async_copy(obuf, dacc.at[didx.at[0]], dsem).wait()

    _ring(topo, sidx, didx, acc, rbufs, gsems, ssems, WINS1, NB1, K1,
          deg_issue, deg_drain)
    plsc.subcore_barrier()
    out_r0 = c * N_PAD + r0
    pltpu.sync_copy(acc.at[pl.ds(r0, STRIPE), :], outp.at[pl.ds(out_r0, STRIPE), :])
    pltpu.sync_copy(dacc.at[pl.ds(r0, STRIPE)], degp.at[pl.ds(out_r0, STRIPE)])


def _seg_body_p2(srcg, dstg, table, z64, outp,
                 sidx, didx, rbufs, acc, gsems, ssems):
    c = lax.axis_index("c")
    s = lax.axis_index("s")
    r0 = s * STRIPE
    pltpu.sync_copy(srcg.at[s], sidx)
    pltpu.sync_copy(dstg.at[s], didx)
    pltpu.sync_copy(z64.at[pl.ds(r0, STRIPE), :], acc.at[pl.ds(r0, STRIPE), :])
    plsc.subcore_barrier()
    _ring(table.at[c], sidx, didx, acc, rbufs, gsems, ssems, WINS2, NB2, K2)
    plsc.subcore_barrier()
    out_r0 = c * N_PAD + r0
    pltpu.sync_copy(acc.at[pl.ds(r0, STRIPE), :], outp.at[pl.ds(out_r0, STRIPE), :])


def _unpack_p1(srcg, dstg, topo, z64, zd, outp, degp, sidx, didx,
               rb0, rb1, rb2, rb3, rb4, rb5, rb6, rb7, obuf, acc, dacc,
               g0, g1, g2, g3, g4, g5, g6, g7,
               s0, s1, s2, s3, s4, s5, s6, s7, dsem):
    _seg_body_p1(srcg, dstg, topo, z64, zd, outp, degp, sidx, didx,
                 (rb0, rb1, rb2, rb3, rb4, rb5, rb6, rb7), obuf, acc, dacc,
                 (g0, g1, g2, g3, g4, g5, g6, g7),
                 (s0, s1, s2, s3, s4, s5, s6, s7), dsem)


def _unpack_p2(srcg, dstg, table, z64, outp, sidx, didx,
               rb0, rb1, rb2, rb3, rb4, acc,
               g0, g1, g2, g3, g4, s0, s1, s2, s3, s4):
    _seg_body_p2(srcg, dstg, table, z64, outp, sidx, didx,
                 (rb0, rb1, rb2, rb3, rb4), acc,
                 (g0, g1, g2, g3, g4), (s0, s1, s2, s3, s4))


_MESH = plsc.VectorSubcoreMesh(core_axis_name="c", subcore_axis_name="s")
_SC_PARAMS = pltpu.CompilerParams(use_tc_tiling_on_sc=False)

_seg_p1 = pl.kernel(
    _unpack_p1,
    out_type=(
        jax.ShapeDtypeStruct((NC * N_PAD, D_TOPO), jnp.float32),
        jax.ShapeDtypeStruct((NC * N_PAD,), jnp.float32),
    ),
    mesh=_MESH,
    scratch_types=[
        pltpu.VMEM((WINS1, WB1), jnp.int32),
        pltpu.VMEM((WINS1, WB1), jnp.int32),
    ] + [pltpu.VMEM((WB1, D_TOPO), jnp.float32)] * NB1 + [
        pltpu.VMEM((WB1,), jnp.float32),
        pltpu.VMEM_SHARED((N_PAD, D_TOPO), jnp.float32),
        pltpu.VMEM_SHARED((N_PAD,), jnp.float32),
    ] + [pltpu.SemaphoreType.DMA] * (2 * NB1 + 1),
    compiler_params=_SC_PARAMS,
)

_seg_p2 = pl.kernel(
    _unpack_p2,
    out_type=jax.ShapeDtypeStruct((NC * N_PAD, D_LAT // 2), jnp.float32),
    mesh=_MESH,
    scratch_types=[
        pltpu.VMEM((WINS2, WB2), jnp.int32),
        pltpu.VMEM((WINS2, WB2), jnp.int32),
    ] + [pltpu.VMEM((WB2, D_LAT // 2), jnp.float32)] * NB2 + [
        pltpu.VMEM_SHARED((N_PAD, D_LAT // 2), jnp.float32),
    ] + [pltpu.SemaphoreType.DMA] * (2 * NB2),
    compiler_params=_SC_PARAMS,
)


def _dense_a_body(p_ref, dg_ref, x_ref, wd_ref, wdn_ref, a_ref, h2_ref, inv_ref):
    a = p_ref[0] + p_ref[1]
    draw = dg_ref[0] + dg_ref[1]
    deg = jnp.maximum(draw, 1.0)
    h = jnp.dot(a, wd_ref[...], preferred_element_type=jnp.float32) / deg
    alpha = a_ref[0, 0]
    lat = jnp.where(h > 0, h, alpha * h)
    w1 = wdn_ref[0:D_FEAT, :]
    w2 = wdn_ref[D_FEAT:, :]
    h2 = (jnp.dot(x_ref[...], w1, preferred_element_type=jnp.float32)
          + jnp.dot(lat, w2, preferred_element_type=jnp.float32))
    h2_ref[0] = h2[:, :D_OUT // 2]
    h2_ref[1] = h2[:, D_OUT // 2:]
    inv_ref[...] = jnp.broadcast_to(1.0 / (deg + 1.0), h2.shape)


def _dense_b_body(p_ref, h2_ref, inv_ref, o_ref):
    o_ref[...] = jnp.concatenate(
        [p_ref[0] + h2_ref[0], p_ref[1] + h2_ref[1]], axis=1) * inv_ref[...]


@jax.jit
def _impl(x, topological_features, edge_index, W_dgi, alpha, W_down):
    f32 = jnp.float32
    src = edge_index[0]
    dst = edge_index[1]
    pe = E_PAD - E
    src_p = jnp.concatenate([src, jnp.zeros((pe,), jnp.int32)])
    dst_p = jnp.concatenate(
        [dst, N + (jnp.arange(pe, dtype=jnp.int32) % (N_PAD - N))])
    srcg1 = src_p.reshape(NW, WINS1, WB1)
    dstg1 = dst_p.reshape(NW, WINS1, WB1)
    srcg2 = src_p.reshape(NS, WINS2, WB2)
    dstg2 = dst_p.reshape(NS, WINS2, WB2)

    z64 = jnp.zeros((N_PAD, D_TOPO), f32)
    zd = jnp.zeros((N_PAD,), f32)

    p1, degp = _seg_p1(srcg1, dstg1, topological_features, z64, zd)

    nblk = N_PAD // R_BLK
    h2s, invb = pl.pallas_call(
        _dense_a_body,
        grid=(nblk,),
        in_specs=[
            pl.BlockSpec((2, R_BLK, D_TOPO), lambda i: (0, i, 0)),
            pl.BlockSpec((2, R_BLK, 1), lambda i: (0, i, 0)),
            pl.BlockSpec((R_BLK, D_FEAT), lambda i: (i, 0)),
            pl.BlockSpec((D_TOPO, D_LAT), lambda i: (0, 0)),
            pl.BlockSpec((D_FEAT + D_LAT, D_OUT), lambda i: (0, 0)),
            pl.BlockSpec((1, 1), lambda i: (0, 0)),
        ],
        out_specs=[
            pl.BlockSpec((2, R_BLK, D_OUT // 2), lambda i: (0, i, 0)),
            pl.BlockSpec((R_BLK, D_OUT), lambda i: (i, 0)),
        ],
        out_shape=[
            jax.ShapeDtypeStruct((2, N_PAD, D_OUT // 2), f32),
            jax.ShapeDtypeStruct((N_PAD, D_OUT), f32),
        ],
    )(p1.reshape(NC, N_PAD, D_TOPO), degp.reshape(NC, N_PAD, 1), x,
      W_dgi, W_down, alpha.reshape(1, 1))

    p2 = _seg_p2(srcg2, dstg2, h2s, z64)

    out = pl.pallas_call(
        _dense_b_body,
        grid=(nblk,),
        in_specs=[
            pl.BlockSpec((2, R_BLK, D_OUT // 2), lambda i: (0, i, 0)),
            pl.BlockSpec((2, R_BLK, D_OUT // 2), lambda i: (0, i, 0)),
            pl.BlockSpec((R_BLK, D_OUT), lambda i: (i, 0)),
        ],
        out_specs=pl.BlockSpec((R_BLK, D_OUT), lambda i: (i, 0)),
        out_shape=jax.ShapeDtypeStruct((N_PAD, D_OUT), f32),
    )(p2.reshape(NC, N_PAD, D_OUT // 2), h2s, invb)

    return out[:N]


def kernel(x, topological_features, edge_index, batch_size, W_dgi, alpha, W_down):
    del batch_size
    return _impl(x, topological_features, edge_index, W_dgi, alpha, W_down)


# speedup vs baseline: 1.0139x; 1.0139x over previous
import jax
import jax.numpy as jnp
from jax import lax
from jax.experimental import pallas as pl
from jax.experimental.pallas import tpu as pltpu
from jax.experimental.pallas import tpu_sc as plsc

N = 10000
E = 320000
D_FEAT = 128
D_TOPO = 64
D_LAT = 128
D_OUT = 128

NC = 2
NS = 16
NW = NC * NS

N_PAD = 10240
STRIPE = N_PAD // NS
WB1 = 128
WINS1 = 80
WB2 = 128
WINS2 = 160
EPT = WB1 * WINS1
E_PAD = NW * EPT
NB1, K1 = 8, 4
NB2, K2 = 5, 2
R_BLK = 1024


def _ring(table, sidx, didx, acc, rbufs, gsems, ssems, wins, nb, k,
          deg_issue=None, deg_drain=None):
    p = nb - k

    def gath(w, b):
        pltpu.async_copy(table.at[sidx.at[w]], rbufs[b], gsems[b])

    def gwait(b):
        pltpu.make_async_copy(table.at[sidx.at[0]], rbufs[b], gsems[b]).wait()

    def scat(w, b):
        pltpu.async_copy(rbufs[b], acc.at[didx.at[w]], ssems[b], add=True)

    def swait(b):
        pltpu.make_async_copy(rbufs[b], acc.at[didx.at[0]], ssems[b]).wait()

    for w in range(p):
        gath(w, w)
    for w in range(k):
        b = w % nb
        gwait(b)
        scat(w, b)
        if deg_issue is not None:
            deg_issue(w)
        gath(w + p, (w + p) % nb)

    def group(g, carry):
        w0 = k + g * nb
        for j in range(nb):
            w = w0 + j
            b = (k + j) % nb
            gwait(b)
            scat(w, b)
            if deg_issue is not None:
                deg_issue(w)
                deg_drain()
            swait(j % nb)
            gath(w + p, j % nb)
        return carry

    lax.fori_loop(0, (wins - nb) // nb, group, 0)
    for t in range(p):
        w = wins - p + t
        b = (k + t) % nb
        gwait(b)
        scat(w, b)
        if deg_issue is not None:
            deg_issue(w)
            deg_drain()
    for b in range(nb):
        swait(b)
    if deg_drain is not None:
        for _ in range(k):
            deg_drain()


def _seg_body_p1(srcg, dstg, topo, z64, zd, outp, degp,
                 sidx, didx, rbufs, obuf, acc, dacc, gsems, ssems, dsem):
    c = lax.axis_index("c")
    s = lax.axis_index("s")
    wid = c * NS + s
    r0 = s * STRIPE
    pltpu.sync_copy(srcg.at[wid], sidx)
    pltpu.sync_copy(dstg.at[wid], didx)
    for i in range(8):
        obuf[pl.ds(i * 16, 16)] = jnp.ones((16,), jnp.float32)
    pltpu.sync_copy(z64.at[pl.ds(r0, STRIPE), :], acc.at[pl.ds(r0, STRIPE), :])
    pltpu.sync_copy(zd.at[pl.ds(r0, STRIPE)], dacc.at[pl.ds(r0, STRIPE)])
    plsc.subcore_barrier()

    def deg_issue(w):
        pltpu.async_copy(obuf, dacc.at[didx.at[w]], dsem, add=True)

    def deg_drain():
        pltpu.make_async_copy(obuf, dacc.at[didx.at[0]], dsem).wait()

    _ring(topo, sidx, didx, acc, rbufs, gsems, ssems, WINS1, NB1, K1,
          deg_issue, deg_drain)
    plsc.subcore_barrier()
    out_r0 = c * N_PAD + r0
    pltpu.sync_copy(acc.at[pl.ds(r0, STRIPE), :], outp.at[pl.ds(out_r0, STRIPE), :])
    pltpu.sync_copy(dacc.at[pl.ds(r0, STRIPE)], degp.at[pl.ds(out_r0, STRIPE)])


def _seg_body_p2(srcg, dstg, table, z64, outp,
                 sidx, didx, rbufs, acc, gsems, ssems):
    c = lax.axis_index("c")
    s = lax.axis_index("s")
    r0 = s * STRIPE
    pltpu.sync_copy(srcg.at[s], sidx)
    pltpu.sync_copy(dstg.at[s], didx)
    pltpu.sync_copy(z64.at[pl.ds(r0, STRIPE), :], acc.at[pl.ds(r0, STRIPE), :])
    plsc.subcore_barrier()
    _ring(table.at[c], sidx, didx, acc, rbufs, gsems, ssems, WINS2, NB2, K2)
    plsc.subcore_barrier()
    out_r0 = c * N_PAD + r0
    pltpu.sync_copy(acc.at[pl.ds(r0, STRIPE), :], outp.at[pl.ds(out_r0, STRIPE), :])


def _unpack_p1(srcg, dstg, topo, z64, zd, outp, degp, sidx, didx,
               rb0, rb1, rb2, rb3, rb4, rb5, rb6, rb7, obuf, acc, dacc,
               g0, g1, g2, g3, g4, g5, g6, g7,
               s0, s1, s2, s3, s4, s5, s6, s7, dsem):
    _seg_body_p1(srcg, dstg, topo, z64, zd, outp, degp, sidx, didx,
                 (rb0, rb1, rb2, rb3, rb4, rb5, rb6, rb7), obuf, acc, dacc,
                 (g0, g1, g2, g3, g4, g5, g6, g7),
                 (s0, s1, s2, s3, s4, s5, s6, s7), dsem)


def _unpack_p2(srcg, dstg, table, z64, outp, sidx, didx,
               rb0, rb1, rb2, rb3, rb4, acc,
               g0, g1, g2, g3, g4, s0, s1, s2, s3, s4):
    _seg_body_p2(srcg, dstg, table, z64, outp, sidx, didx,
                 (rb0, rb1, rb2, rb3, rb4), acc,
                 (g0, g1, g2, g3, g4), (s0, s1, s2, s3, s4))


_MESH = plsc.VectorSubcoreMesh(core_axis_name="c", subcore_axis_name="s")
_SC_PARAMS = pltpu.CompilerParams(use_tc_tiling_on_sc=False)

_seg_p1 = pl.kernel(
    _unpack_p1,
    out_type=(
        jax.ShapeDtypeStruct((NC * N_PAD, D_TOPO), jnp.float32),
        jax.ShapeDtypeStruct((NC * N_PAD,), jnp.float32),
    ),
    mesh=_MESH,
    scratch_types=[
        pltpu.VMEM((WINS1, WB1), jnp.int32),
        pltpu.VMEM((WINS1, WB1), jnp.int32),
    ] + [pltpu.VMEM((WB1, D_TOPO), jnp.float32)] * NB1 + [
        pltpu.VMEM((WB1,), jnp.float32),
        pltpu.VMEM_SHARED((N_PAD, D_TOPO), jnp.float32),
        pltpu.VMEM_SHARED((N_PAD,), jnp.float32),
    ] + [pltpu.SemaphoreType.DMA] * (2 * NB1 + 1),
    compiler_params=_SC_PARAMS,
)

_seg_p2 = pl.kernel(
    _unpack_p2,
    out_type=jax.ShapeDtypeStruct((NC * N_PAD, D_LAT // 2), jnp.float32),
    mesh=_MESH,
    scratch_types=[
        pltpu.VMEM((WINS2, WB2), jnp.int32),
        pltpu.VMEM((WINS2, WB2), jnp.int32),
    ] + [pltpu.VMEM((WB2, D_LAT // 2), jnp.float32)] * NB2 + [
        pltpu.VMEM_SHARED((N_PAD, D_LAT // 2), jnp.float32),
    ] + [pltpu.SemaphoreType.DMA] * (2 * NB2),
    compiler_params=_SC_PARAMS,
)


def _dense_a_body(p_ref, dg_ref, x_ref, wd_ref, wdn_ref, a_ref, h2_ref, inv_ref):
    a = p_ref[0] + p_ref[1]
    draw = dg_ref[0] + dg_ref[1]
    deg = jnp.maximum(draw, 1.0)
    h = jnp.dot(a, wd_ref[...], preferred_element_type=jnp.float32) / deg
    alpha = a_ref[0, 0]
    lat = jnp.where(h > 0, h, alpha * h)
    w1 = wdn_ref[0:D_FEAT, :]
    w2 = wdn_ref[D_FEAT:, :]
    h2 = (jnp.dot(x_ref[...], w1, preferred_element_type=jnp.float32)
          + jnp.dot(lat, w2, preferred_element_type=jnp.float32))
    h2_ref[0] = h2[:, :D_OUT // 2]
    h2_ref[1] = h2[:, D_OUT // 2:]
    inv_ref[...] = jnp.broadcast_to(1.0 / (deg + 1.0), h2.shape)


def _dense_b_body(p_ref, h2_ref, inv_ref, o_ref):
    o_ref[...] = jnp.concatenate(
        [p_ref[0] + h2_ref[0], p_ref[1] + h2_ref[1]], axis=1) * inv_ref[...]


@jax.jit
def _impl(x, topological_features, edge_index, W_dgi, alpha, W_down):
    f32 = jnp.float32
    src = edge_index[0]
    dst = edge_index[1]
    pe = E_PAD - E
    src_p = jnp.concatenate([src, jnp.zeros((pe,), jnp.int32)])
    dst_p = jnp.concatenate(
        [dst, N + (jnp.arange(pe, dtype=jnp.int32) % (N_PAD - N))])
    srcg1 = src_p.reshape(NW, WINS1, WB1)
    dstg1 = dst_p.reshape(NW, WINS1, WB1)
    srcg2 = src_p.reshape(NS, WINS2, WB2)
    dstg2 = dst_p.reshape(NS, WINS2, WB2)

    z64 = jnp.zeros((N_PAD, D_TOPO), f32)
    zd = jnp.zeros((N_PAD,), f32)

    p1, degp = _seg_p1(srcg1, dstg1, topological_features, z64, zd)

    nblk = N_PAD // R_BLK
    h2s, invb = pl.pallas_call(
        _dense_a_body,
        grid=(nblk,),
        in_specs=[
            pl.BlockSpec((2, R_BLK, D_TOPO), lambda i: (0, i, 0)),
            pl.BlockSpec((2, R_BLK, 1), lambda i: (0, i, 0)),
            pl.BlockSpec((R_BLK, D_FEAT), lambda i: (i, 0)),
            pl.BlockSpec((D_TOPO, D_LAT), lambda i: (0, 0)),
            pl.BlockSpec((D_FEAT + D_LAT, D_OUT), lambda i: (0, 0)),
            pl.BlockSpec((1, 1), lambda i: (0, 0)),
        ],
        out_specs=[
            pl.BlockSpec((2, R_BLK, D_OUT // 2), lambda i: (0, i, 0)),
            pl.BlockSpec((R_BLK, D_OUT), lambda i: (i, 0)),
        ],
        out_shape=[
            jax.ShapeDtypeStruct((2, N_PAD, D_OUT // 2), f32),
            jax.ShapeDtypeStruct((N_PAD, D_OUT), f32),
        ],
    )(p1.reshape(NC, N_PAD, D_TOPO), degp.reshape(NC, N_PAD, 1), x,
      W_dgi, W_down, alpha.reshape(1, 1))

    p2 = _seg_p2(srcg2, dstg2, h2s, z64)

    out = pl.pallas_call(
        _dense_b_body,
        grid=(nblk,),
        in_specs=[
            pl.BlockSpec((2, R_BLK, D_OUT // 2), lambda i: (0, i, 0)),
            pl.BlockSpec((2, R_BLK, D_OUT // 2), lambda i: (0, i, 0)),
            pl.BlockSpec((R_BLK, D_OUT), lambda i: (i, 0)),
        ],
        out_specs=pl.BlockSpec((R_BLK, D_OUT), lambda i: (i, 0)),
        out_shape=jax.ShapeDtypeStruct((N_PAD, D_OUT), f32),
    )(p2.reshape(NC, N_PAD, D_OUT // 2), h2s, invb)

    return out[:N]


def kernel(x, topological_features, edge_index, batch_size, W_dgi, alpha, W_down):
    del batch_size
    return _impl(x, topological_features, edge_index, W_dgi, alpha, W_down)


# speedup vs baseline: 1.1569x; 1.1410x over previous
import jax
import jax.numpy as jnp
from jax import lax
from jax.experimental import pallas as pl
from jax.experimental.pallas import tpu as pltpu
from jax.experimental.pallas import tpu_sc as plsc

N = 10000
E = 320000
D_FEAT = 128
D_TOPO = 64
D_LAT = 128
D_OUT = 128

NC = 2
NS = 16
NW = NC * NS

N_PAD = 10240
STRIPE = N_PAD // NS
WB1 = 128
WINS1 = 80
WB2 = 128
WINS2 = 160
EPT = WB1 * WINS1
E_PAD = NW * EPT
NB1, K1 = 8, 4
NB2, K2 = 5, 2
R_BLK = 1024


def _ring(table, sidx, didx, acc, rbufs, gsems, ssems, wins, nb, k,
          deg_issue=None, deg_drain=None):
    p = nb - k

    def gath(w, b):
        pltpu.async_copy(table.at[sidx.at[w]], rbufs[b], gsems[b])

    def gwait(b):
        pltpu.make_async_copy(table.at[sidx.at[0]], rbufs[b], gsems[b]).wait()

    def scat(w, b):
        pltpu.async_copy(rbufs[b], acc.at[didx.at[w]], ssems[b], add=True)

    def swait(b):
        pltpu.make_async_copy(rbufs[b], acc.at[didx.at[0]], ssems[b]).wait()

    for w in range(p):
        gath(w, w)
    for w in range(k):
        b = w % nb
        gwait(b)
        scat(w, b)
        if deg_issue is not None:
            deg_issue(w)
        gath(w + p, (w + p) % nb)

    def group(g, carry):
        w0 = k + g * nb
        for j in range(nb):
            w = w0 + j
            b = (k + j) % nb
            gwait(b)
            scat(w, b)
            if deg_issue is not None:
                deg_issue(w)
                deg_drain()
            swait(j % nb)
            gath(w + p, j % nb)
        return carry

    lax.fori_loop(0, (wins - nb) // nb, group, 0)
    for t in range(p):
        w = wins - p + t
        b = (k + t) % nb
        gwait(b)
        scat(w, b)
        if deg_issue is not None:
            deg_issue(w)
            deg_drain()
    for b in range(nb):
        swait(b)
    if deg_drain is not None:
        for _ in range(k):
            deg_drain()


def _seg_body_p1(srcg, dstg, topo, z64, zd, outp, degp,
                 sidx, didx, rbufs, obuf, acc, dacc, gsems, ssems, dsem):
    c = lax.axis_index("c")
    s = lax.axis_index("s")
    wid = c * NS + s
    r0 = s * STRIPE
    pltpu.sync_copy(srcg.at[wid], sidx)
    pltpu.sync_copy(dstg.at[wid], didx)
    for i in range(8):
        obuf[pl.ds(i * 16, 16)] = jnp.ones((16,), jnp.float32)
    pltpu.sync_copy(z64.at[pl.ds(r0, STRIPE), :], acc.at[pl.ds(r0, STRIPE), :])
    pltpu.sync_copy(zd.at[pl.ds(r0, STRIPE)], dacc.at[pl.ds(r0, STRIPE)])
    plsc.subcore_barrier()

    def deg_issue(w):
        pltpu.async_copy(obuf, dacc.at[didx.at[w]], dsem, add=True)

    def deg_drain():
        pltpu.make_async_copy(obuf, dacc.at[didx.at[0]], dsem).wait()

    _ring(topo, sidx, didx, acc, rbufs, gsems, ssems, WINS1, NB1, K1,
          deg_issue, deg_drain)
    plsc.subcore_barrier()
    out_r0 = c * N_PAD + r0
    pltpu.sync_copy(acc.at[pl.ds(r0, STRIPE), :], outp.at[pl.ds(out_r0, STRIPE), :])
    pltpu.sync_copy(dacc.at[pl.ds(r0, STRIPE)], degp.at[pl.ds(out_r0, STRIPE)])


def _seg_body_p2(srcg, dstg, table, z64, outp,
                 sidx, didx, rbufs, acc, gsems, ssems):
    c = lax.axis_index("c")
    s = lax.axis_index("s")
    r0 = s * STRIPE
    pltpu.sync_copy(srcg.at[s], sidx)
    pltpu.sync_copy(dstg.at[s], didx)
    pltpu.sync_copy(z64.at[pl.ds(r0, STRIPE), :], acc.at[pl.ds(r0, STRIPE), :])
    plsc.subcore_barrier()
    _ring(table.at[c], sidx, didx, acc, rbufs, gsems, ssems, WINS2, NB2, K2)
    plsc.subcore_barrier()
    out_r0 = c * N_PAD + r0
    pltpu.sync_copy(acc.at[pl.ds(r0, STRIPE), :], outp.at[pl.ds(out_r0, STRIPE), :])


def _unpack_p1(srcg, dstg, topo, z64, zd, outp, degp, sidx, didx,
               rb0, rb1, rb2, rb3, rb4, rb5, rb6, rb7, obuf, acc, dacc,
               g0, g1, g2, g3, g4, g5, g6, g7,
               s0, s1, s2, s3, s4, s5, s6, s7, dsem):
    _seg_body_p1(srcg, dstg, topo, z64, zd, outp, degp, sidx, didx,
                 (rb0, rb1, rb2, rb3, rb4, rb5, rb6, rb7), obuf, acc, dacc,
                 (g0, g1, g2, g3, g4, g5, g6, g7),
                 (s0, s1, s2, s3, s4, s5, s6, s7), dsem)


def _unpack_p2(srcg, dstg, table, z64, outp, sidx, didx,
               rb0, rb1, rb2, rb3, rb4, acc,
               g0, g1, g2, g3, g4, s0, s1, s2, s3, s4):
    _seg_body_p2(srcg, dstg, table, z64, outp, sidx, didx,
                 (rb0, rb1, rb2, rb3, rb4), acc,
                 (g0, g1, g2, g3, g4), (s0, s1, s2, s3, s4))


_MESH = plsc.VectorSubcoreMesh(core_axis_name="c", subcore_axis_name="s")
_SC_PARAMS = pltpu.CompilerParams(use_tc_tiling_on_sc=False)

_seg_p1 = pl.kernel(
    _unpack_p1,
    out_type=(
        jax.ShapeDtypeStruct((NC * N_PAD, D_TOPO), jnp.float32),
        jax.ShapeDtypeStruct((NC * N_PAD,), jnp.float32),
    ),
    mesh=_MESH,
    scratch_types=[
        pltpu.VMEM((WINS1, WB1), jnp.int32),
        pltpu.VMEM((WINS1, WB1), jnp.int32),
    ] + [pltpu.VMEM((WB1, D_TOPO), jnp.float32)] * NB1 + [
        pltpu.VMEM((WB1,), jnp.float32),
        pltpu.VMEM_SHARED((N_PAD, D_TOPO), jnp.float32),
        pltpu.VMEM_SHARED((N_PAD,), jnp.float32),
    ] + [pltpu.SemaphoreType.DMA] * (2 * NB1 + 1),
    compiler_params=_SC_PARAMS,
)

_seg_p2 = pl.kernel(
    _unpack_p2,
    out_type=jax.ShapeDtypeStruct((NC * N_PAD, D_LAT // 2), jnp.float32),
    mesh=_MESH,
    scratch_types=[
        pltpu.VMEM((WINS2, WB2), jnp.int32),
        pltpu.VMEM((WINS2, WB2), jnp.int32),
    ] + [pltpu.VMEM((WB2, D_LAT // 2), jnp.float32)] * NB2 + [
        pltpu.VMEM_SHARED((N_PAD, D_LAT // 2), jnp.float32),
    ] + [pltpu.SemaphoreType.DMA] * (2 * NB2),
    compiler_params=_SC_PARAMS,
)


def _dense_a_body(p_ref, dg_ref, x_ref, wd_ref, wdn_ref, a_ref, h2_ref, inv_ref):
    a = p_ref[0] + p_ref[1]
    draw = dg_ref[0] + dg_ref[1]
    deg = jnp.maximum(draw, 1.0)
    h = jnp.dot(a, wd_ref[...], preferred_element_type=jnp.float32) / deg
    alpha = a_ref[0, 0]
    lat = jnp.where(h > 0, h, alpha * h)
    w1 = wdn_ref[0:D_FEAT, :]
    w2 = wdn_ref[D_FEAT:, :]
    h2 = (jnp.dot(x_ref[...], w1, preferred_element_type=jnp.float32)
          + jnp.dot(lat, w2, preferred_element_type=jnp.float32))
    h2_ref[0] = h2[:, :D_OUT // 2]
    h2_ref[1] = h2[:, D_OUT // 2:]
    inv_ref[...] = jnp.broadcast_to(1.0 / (deg + 1.0), h2.shape)


def _dense_b_body(p_ref, h2_ref, inv_ref, o_ref):
    o_ref[...] = jnp.concatenate(
        [p_ref[0] + h2_ref[0], p_ref[1] + h2_ref[1]], axis=1) * inv_ref[...]


@jax.jit
def _impl(x, topological_features, edge_index, W_dgi, alpha, W_down):
    f32 = jnp.float32
    src = edge_index[0]
    dst = edge_index[1]
    pe = E_PAD - E
    src_p = jnp.concatenate([src, jnp.zeros((pe,), jnp.int32)])
    dst_p = jnp.concatenate(
        [dst, N + (jnp.arange(pe, dtype=jnp.int32) % (N_PAD - N))])
    srcg1 = src_p.reshape(NW, WINS1, WB1)
    dstg1 = dst_p.reshape(NW, WINS1, WB1)
    srcg2 = src_p.reshape(NS, WINS2, WB2)
    dstg2 = dst_p.reshape(NS, WINS2, WB2)

    z64 = jnp.zeros((N_PAD, D_TOPO), f32)
    zd = jnp.zeros((N_PAD,), f32)

    p1, degp = _seg_p1(srcg1, dstg1, topological_features, z64, zd)

    x_pad = jnp.pad(x, ((0, N_PAD - N), (0, 0)))
    nblk = N_PAD // R_BLK
    h2s, invb = pl.pallas_call(
        _dense_a_body,
        grid=(nblk,),
        in_specs=[
            pl.BlockSpec((2, R_BLK, D_TOPO), lambda i: (0, i, 0)),
            pl.BlockSpec((2, R_BLK, 1), lambda i: (0, i, 0)),
            pl.BlockSpec((R_BLK, D_FEAT), lambda i: (i, 0)),
            pl.BlockSpec((D_TOPO, D_LAT), lambda i: (0, 0)),
            pl.BlockSpec((D_FEAT + D_LAT, D_OUT), lambda i: (0, 0)),
            pl.BlockSpec((1, 1), lambda i: (0, 0)),
        ],
        out_specs=[
            pl.BlockSpec((2, R_BLK, D_OUT // 2), lambda i: (0, i, 0)),
            pl.BlockSpec((R_BLK, D_OUT), lambda i: (i, 0)),
        ],
        out_shape=[
            jax.ShapeDtypeStruct((2, N_PAD, D_OUT // 2), f32),
            jax.ShapeDtypeStruct((N_PAD, D_OUT), f32),
        ],
    )(p1.reshape(NC, N_PAD, D_TOPO), degp.reshape(NC, N_PAD, 1), x_pad,
      W_dgi, W_down, alpha.reshape(1, 1))

    p2 = _seg_p2(srcg2, dstg2, h2s, z64)

    out = pl.pallas_call(
        _dense_b_body,
        grid=(nblk,),
        in_specs=[
            pl.BlockSpec((2, R_BLK, D_OUT // 2), lambda i: (0, i, 0)),
            pl.BlockSpec((2, R_BLK, D_OUT // 2), lambda i: (0, i, 0)),
            pl.BlockSpec((R_BLK, D_OUT), lambda i: (i, 0)),
        ],
        out_specs=pl.BlockSpec((R_BLK, D_OUT), lambda i: (i, 0)),
        out_shape=jax.ShapeDtypeStruct((N_PAD, D_OUT), f32),
    )(p2.reshape(NC, N_PAD, D_OUT // 2), h2s, invb)

    return out[:N]


def kernel(x, topological_features, edge_index, batch_size, W_dgi, alpha, W_down):
    del batch_size
    return _impl(x, topological_features, edge_index, W_dgi, alpha, W_down)


# speedup vs baseline: 1.1579x; 1.0009x over previous
import jax
import jax.numpy as jnp
from jax import lax
from jax.experimental import pallas as pl
from jax.experimental.pallas import tpu as pltpu
from jax.experimental.pallas import tpu_sc as plsc

N = 10000
E = 320000
D_FEAT = 128
D_TOPO = 64
D_LAT = 128
D_OUT = 128

NC = 2
NS = 16
NW = NC * NS

N_PAD = 10240
STRIPE = N_PAD // NS
WB1 = 128
WINS1 = 80
WB2 = 128
WINS2 = 160
EPT = WB1 * WINS1
E_PAD = NW * EPT
NB1, K1 = 8, 2
NB2, K2 = 5, 2
R_BLK = 1024


def _ring(table, sidx, didx, acc, rbufs, gsems, ssems, wins, nb, k,
          deg_issue=None, deg_drain=None):
    p = nb - k

    def gath(w, b):
        pltpu.async_copy(table.at[sidx.at[w]], rbufs[b], gsems[b])

    def gwait(b):
        pltpu.make_async_copy(table.at[sidx.at[0]], rbufs[b], gsems[b]).wait()

    def scat(w, b):
        pltpu.async_copy(rbufs[b], acc.at[didx.at[w]], ssems[b], add=True)

    def swait(b):
        pltpu.make_async_copy(rbufs[b], acc.at[didx.at[0]], ssems[b]).wait()

    for w in range(p):
        gath(w, w)
    for w in range(k):
        b = w % nb
        gwait(b)
        scat(w, b)
        if deg_issue is not None:
            deg_issue(w)
        gath(w + p, (w + p) % nb)

    def group(g, carry):
        w0 = k + g * nb
        for j in range(nb):
            w = w0 + j
            b = (k + j) % nb
            gwait(b)
            scat(w, b)
            if deg_issue is not None:
                deg_issue(w)
                deg_drain()
            swait(j % nb)
            gath(w + p, j % nb)
        return carry

    lax.fori_loop(0, (wins - nb) // nb, group, 0)
    for t in range(p):
        w = wins - p + t
        b = (k + t) % nb
        gwait(b)
        scat(w, b)
        if deg_issue is not None:
            deg_issue(w)
            deg_drain()
    for b in range(nb):
        swait(b)
    if deg_drain is not None:
        for _ in range(k):
            deg_drain()


def _seg_body_p1(srcg, dstg, topo, z64, zd, outp, degp,
                 sidx, didx, rbufs, obuf, acc, dacc, gsems, ssems, dsem):
    c = lax.axis_index("c")
    s = lax.axis_index("s")
    wid = c * NS + s
    r0 = s * STRIPE
    pltpu.sync_copy(srcg.at[wid], sidx)
    pltpu.sync_copy(dstg.at[wid], didx)
    for i in range(8):
        obuf[pl.ds(i * 16, 16)] = jnp.ones((16,), jnp.float32)
    pltpu.sync_copy(z64.at[pl.ds(r0, STRIPE), :], acc.at[pl.ds(r0, STRIPE), :])
    pltpu.sync_copy(zd.at[pl.ds(r0, STRIPE)], dacc.at[pl.ds(r0, STRIPE)])
    plsc.subcore_barrier()

    def deg_issue(w):
        pltpu.async_copy(obuf, dacc.at[didx.at[w]], dsem, add=True)

    def deg_drain():
        pltpu.make_async_copy(obuf, dacc.at[didx.at[0]], dsem).wait()

    _ring(topo, sidx, didx, acc, rbufs, gsems, ssems, WINS1, NB1, K1,
          deg_issue, deg_drain)
    plsc.subcore_barrier()
    out_r0 = c * N_PAD + r0
    pltpu.sync_copy(acc.at[pl.ds(r0, STRIPE), :], outp.at[pl.ds(out_r0, STRIPE), :])
    pltpu.sync_copy(dacc.at[pl.ds(r0, STRIPE)], degp.at[pl.ds(out_r0, STRIPE)])


def _seg_body_p2(srcg, dstg, table, z64, outp,
                 sidx, didx, rbufs, acc, gsems, ssems):
    c = lax.axis_index("c")
    s = lax.axis_index("s")
    r0 = s * STRIPE
    pltpu.sync_copy(srcg.at[s], sidx)
    pltpu.sync_copy(dstg.at[s], didx)
    pltpu.sync_copy(z64.at[pl.ds(r0, STRIPE), :], acc.at[pl.ds(r0, STRIPE), :])
    plsc.subcore_barrier()
    _ring(table.at[c], sidx, didx, acc, rbufs, gsems, ssems, WINS2, NB2, K2)
    plsc.subcore_barrier()
    out_r0 = c * N_PAD + r0
    pltpu.sync_copy(acc.at[pl.ds(r0, STRIPE), :], outp.at[pl.ds(out_r0, STRIPE), :])


def _unpack_p1(srcg, dstg, topo, z64, zd, outp, degp, sidx, didx,
               rb0, rb1, rb2, rb3, rb4, rb5, rb6, rb7, obuf, acc, dacc,
               g0, g1, g2, g3, g4, g5, g6, g7,
               s0, s1, s2, s3, s4, s5, s6, s7, dsem):
    _seg_body_p1(srcg, dstg, topo, z64, zd, outp, degp, sidx, didx,
                 (rb0, rb1, rb2, rb3, rb4, rb5, rb6, rb7), obuf, acc, dacc,
                 (g0, g1, g2, g3, g4, g5, g6, g7),
                 (s0, s1, s2, s3, s4, s5, s6, s7), dsem)


def _unpack_p2(srcg, dstg, table, z64, outp, sidx, didx,
               rb0, rb1, rb2, rb3, rb4, acc,
               g0, g1, g2, g3, g4, s0, s1, s2, s3, s4):
    _seg_body_p2(srcg, dstg, table, z64, outp, sidx, didx,
                 (rb0, rb1, rb2, rb3, rb4), acc,
                 (g0, g1, g2, g3, g4), (s0, s1, s2, s3, s4))


_MESH = plsc.VectorSubcoreMesh(core_axis_name="c", subcore_axis_name="s")
_SC_PARAMS = pltpu.CompilerParams(use_tc_tiling_on_sc=False)

_seg_p1 = pl.kernel(
    _unpack_p1,
    out_type=(
        jax.ShapeDtypeStruct((NC * N_PAD, D_TOPO), jnp.float32),
        jax.ShapeDtypeStruct((NC * N_PAD,), jnp.float32),
    ),
    mesh=_MESH,
    scratch_types=[
        pltpu.VMEM((WINS1, WB1), jnp.int32),
        pltpu.VMEM((WINS1, WB1), jnp.int32),
    ] + [pltpu.VMEM((WB1, D_TOPO), jnp.float32)] * NB1 + [
        pltpu.VMEM((WB1,), jnp.float32),
        pltpu.VMEM_SHARED((N_PAD, D_TOPO), jnp.float32),
        pltpu.VMEM_SHARED((N_PAD,), jnp.float32),
    ] + [pltpu.SemaphoreType.DMA] * (2 * NB1 + 1),
    compiler_params=_SC_PARAMS,
)

_seg_p2 = pl.kernel(
    _unpack_p2,
    out_type=jax.ShapeDtypeStruct((NC * N_PAD, D_LAT // 2), jnp.float32),
    mesh=_MESH,
    scratch_types=[
        pltpu.VMEM((WINS2, WB2), jnp.int32),
        pltpu.VMEM((WINS2, WB2), jnp.int32),
    ] + [pltpu.VMEM((WB2, D_LAT // 2), jnp.float32)] * NB2 + [
        pltpu.VMEM_SHARED((N_PAD, D_LAT // 2), jnp.float32),
    ] + [pltpu.SemaphoreType.DMA] * (2 * NB2),
    compiler_params=_SC_PARAMS,
)


def _dense_a_body(p_ref, dg_ref, x_ref, wd_ref, wdn_ref, a_ref, h2_ref, inv_ref):
    a = p_ref[0] + p_ref[1]
    draw = dg_ref[0] + dg_ref[1]
    deg = jnp.maximum(draw, 1.0)
    h = jnp.dot(a, wd_ref[...], preferred_element_type=jnp.float32) / deg
    alpha = a_ref[0, 0]
    lat = jnp.where(h > 0, h, alpha * h)
    w1 = wdn_ref[0:D_FEAT, :]
    w2 = wdn_ref[D_FEAT:, :]
    h2 = (jnp.dot(x_ref[...], w1, preferred_element_type=jnp.float32)
          + jnp.dot(lat, w2, preferred_element_type=jnp.float32))
    h2_ref[0] = h2[:, :D_OUT // 2]
    h2_ref[1] = h2[:, D_OUT // 2:]
    inv_ref[...] = jnp.broadcast_to(1.0 / (deg + 1.0), h2.shape)


def _dense_b_body(p_ref, h2_ref, inv_ref, o_ref):
    o_ref[...] = jnp.concatenate(
        [p_ref[0] + h2_ref[0], p_ref[1] + h2_ref[1]], axis=1) * inv_ref[...]


@jax.jit
def _impl(x, topological_features, edge_index, W_dgi, alpha, W_down):
    f32 = jnp.float32
    src = edge_index[0]
    dst = edge_index[1]
    pe = E_PAD - E
    src_p = jnp.concatenate([src, jnp.zeros((pe,), jnp.int32)])
    dst_p = jnp.concatenate(
        [dst, N + (jnp.arange(pe, dtype=jnp.int32) % (N_PAD - N))])
    srcg1 = src_p.reshape(NW, WINS1, WB1)
    dstg1 = dst_p.reshape(NW, WINS1, WB1)
    srcg2 = src_p.reshape(NS, WINS2, WB2)
    dstg2 = dst_p.reshape(NS, WINS2, WB2)

    z64 = jnp.zeros((N_PAD, D_TOPO), f32)
    zd = jnp.zeros((N_PAD,), f32)

    p1, degp = _seg_p1(srcg1, dstg1, topological_features, z64, zd)

    x_pad = jnp.pad(x, ((0, N_PAD - N), (0, 0)))
    nblk = N_PAD // R_BLK
    h2s, invb = pl.pallas_call(
        _dense_a_body,
        grid=(nblk,),
        in_specs=[
            pl.BlockSpec((2, R_BLK, D_TOPO), lambda i: (0, i, 0)),
            pl.BlockSpec((2, R_BLK, 1), lambda i: (0, i, 0)),
            pl.BlockSpec((R_BLK, D_FEAT), lambda i: (i, 0)),
            pl.BlockSpec((D_TOPO, D_LAT), lambda i: (0, 0)),
            pl.BlockSpec((D_FEAT + D_LAT, D_OUT), lambda i: (0, 0)),
            pl.BlockSpec((1, 1), lambda i: (0, 0)),
        ],
        out_specs=[
            pl.BlockSpec((2, R_BLK, D_OUT // 2), lambda i: (0, i, 0)),
            pl.BlockSpec((R_BLK, D_OUT), lambda i: (i, 0)),
        ],
        out_shape=[
            jax.ShapeDtypeStruct((2, N_PAD, D_OUT // 2), f32),
            jax.ShapeDtypeStruct((N_PAD, D_OUT), f32),
        ],
    )(p1.reshape(NC, N_PAD, D_TOPO), degp.reshape(NC, N_PAD, 1), x_pad,
      W_dgi, W_down, alpha.reshape(1, 1))

    p2 = _seg_p2(srcg2, dstg2, h2s, z64)

    out = pl.pallas_call(
        _dense_b_body,
        grid=(nblk,),
        in_specs=[
            pl.BlockSpec((2, R_BLK, D_OUT // 2), lambda i: (0, i, 0)),
            pl.BlockSpec((2, R_BLK, D_OUT // 2), lambda i: (0, i, 0)),
            pl.BlockSpec((R_BLK, D_OUT), lambda i: (i, 0)),
        ],
        out_specs=pl.BlockSpec((R_BLK, D_OUT), lambda i: (i, 0)),
        out_shape=jax.ShapeDtypeStruct((N_PAD, D_OUT), f32),
    )(p2.reshape(NC, N_PAD, D_OUT // 2), h2s, invb)

    return out[:N]


def kernel(x, topological_features, edge_index, batch_size, W_dgi, alpha, W_down):
    del batch_size
    return _impl(x, topological_features, edge_index, W_dgi, alpha, W_down)
